# baseline (speedup 1.0000x reference)
import functools

import jax
import jax.numpy as jnp
from jax.experimental import pallas as pl


_VAL_TAU = 0.9
_VAL_P = 0.8


def _gate_kernel(patch_ref, w_ref, cb_ref, fcw_ref, fcb_ref, out_ref):
    bb = patch_ref.shape[0]
    s = patch_ref.shape[2]
    x = patch_ref[...]
    xt = jnp.swapaxes(x, 1, 2)
    xt = xt.reshape(bb * s, x.shape[1])
    y = jnp.dot(xt, w_ref[...], preferred_element_type=jnp.float32)
    y = jax.nn.relu(y + cb_ref[...])
    pooled = jnp.mean(y.reshape(bb, s, y.shape[1]), axis=1)
    logits = jnp.dot(pooled, fcw_ref[...],
                     preferred_element_type=jnp.float32) + fcb_ref[...]
    z = logits / _VAL_TAU
    z = z - jnp.max(z, axis=-1, keepdims=True)
    e = jnp.exp(z)
    probs = e / jnp.sum(e, axis=-1, keepdims=True)

    n_e = probs.shape[-1]
    pj = probs[:, :, None]
    pe = probs[:, None, :]
    jj = jax.lax.broadcasted_iota(jnp.int32, (1, n_e, n_e), 1)
    ee = jax.lax.broadcasted_iota(jnp.int32, (1, n_e, n_e), 2)
    ahead_or_self = (pj > pe) | ((pj == pe) & (jj <= ee))
    contrib = jnp.where(ahead_or_self, pj, 0.0)
    s_incl = jnp.sum(contrib, axis=1)
    keep = (s_incl <= _VAL_P) | (s_incl - probs == 0.0)
    masked = jnp.where(keep, probs, 0.0)
    denom = jnp.clip(jnp.sum(masked, axis=-1, keepdims=True), 1e-10, None)
    out_ref[...] = masked / denom


def kernel(patch, conv_w, conv_b, fc_w, fc_b, layer_idx, threshold):
    del layer_idx, threshold
    b, c, h, w = patch.shape
    s = h * w
    n_e = fc_w.shape[0]
    patch3 = patch.reshape(b, c, s)
    w_mat = conv_w.T
    cb = conv_b.reshape(1, -1)
    fcw = fc_w.T
    fcb = fc_b.reshape(1, -1)

    bb = 64
    grid = (b // bb,)
    out = pl.pallas_call(
        _gate_kernel,
        grid=grid,
        in_specs=[
            pl.BlockSpec((bb, c, s), lambda i: (i, 0, 0)),
            pl.BlockSpec((c, conv_w.shape[0]), lambda i: (0, 0)),
            pl.BlockSpec((1, conv_w.shape[0]), lambda i: (0, 0)),
            pl.BlockSpec((conv_w.shape[0], n_e), lambda i: (0, 0)),
            pl.BlockSpec((1, n_e), lambda i: (0, 0)),
        ],
        out_specs=pl.BlockSpec((bb, n_e), lambda i: (i, 0)),
        out_shape=jax.ShapeDtypeStruct((b, n_e), jnp.float32),
    )(patch3, w_mat, cb, fcw, fcb)
    return out


# speedup vs baseline: 1.4142x; 1.4142x over previous
import functools

import jax
import jax.numpy as jnp
from jax.experimental import pallas as pl


_VAL_TAU = 0.9
_VAL_P = 0.8


def _gate_kernel(patch_ref, we_ref, wo_ref, cb_ref, fcw_ref, fcb_ref, out_ref):
    bb = patch_ref.shape[0]
    s = 64
    ch = patch_ref.shape[1] // (2 * s)
    u3 = patch_ref[...].reshape(bb, ch, 2 * s)
    v = jnp.swapaxes(u3, 1, 2)
    e = v[:, :s, :].reshape(bb * s, ch)
    o = v[:, s:, :].reshape(bb * s, ch)
    y = (jnp.dot(e, we_ref[...], preferred_element_type=jnp.float32)
         + jnp.dot(o, wo_ref[...], preferred_element_type=jnp.float32))
    y = jax.nn.relu(y + cb_ref[...])
    pooled = jnp.mean(y.reshape(bb, s, y.shape[1]), axis=1)
    logits = jnp.dot(pooled, fcw_ref[...],
                     preferred_element_type=jnp.float32) + fcb_ref[...]
    z = logits / _VAL_TAU
    z = z - jnp.max(z, axis=-1, keepdims=True)
    e = jnp.exp(z)
    probs = e / jnp.sum(e, axis=-1, keepdims=True)

    n_e = probs.shape[-1]
    pj = probs[:, :, None]
    pe = probs[:, None, :]
    jj = jax.lax.broadcasted_iota(jnp.int32, (1, n_e, n_e), 1)
    ee = jax.lax.broadcasted_iota(jnp.int32, (1, n_e, n_e), 2)
    ahead_or_self = (pj > pe) | ((pj == pe) & (jj <= ee))
    contrib = jnp.where(ahead_or_self, pj, 0.0)
    s_incl = jnp.sum(contrib, axis=1)
    keep = (s_incl <= _VAL_P) | (s_incl - probs == 0.0)
    masked = jnp.where(keep, probs, 0.0)
    denom = jnp.clip(jnp.sum(masked, axis=-1, keepdims=True), 1e-10, None)
    out_ref[...] = masked / denom


def kernel(patch, conv_w, conv_b, fc_w, fc_b, layer_idx, threshold):
    del layer_idx, threshold
    b, c, h, w = patch.shape
    s = h * w
    n_e = fc_w.shape[0]
    patch2 = patch.reshape(b, c * s)
    w_mat = conv_w.T
    w_even = w_mat[0::2]
    w_odd = w_mat[1::2]
    cb = conv_b.reshape(1, -1)
    fcw = fc_w.T
    fcb = fc_b.reshape(1, -1)

    bb = 64
    grid = (b // bb,)
    n_o = conv_w.shape[0]
    out = pl.pallas_call(
        _gate_kernel,
        grid=grid,
        in_specs=[
            pl.BlockSpec((bb, c * s), lambda i: (i, 0)),
            pl.BlockSpec((c // 2, n_o), lambda i: (0, 0)),
            pl.BlockSpec((c // 2, n_o), lambda i: (0, 0)),
            pl.BlockSpec((1, n_o), lambda i: (0, 0)),
            pl.BlockSpec((n_o, n_e), lambda i: (0, 0)),
            pl.BlockSpec((1, n_e), lambda i: (0, 0)),
        ],
        out_specs=pl.BlockSpec((bb, n_e), lambda i: (i, 0)),
        out_shape=jax.ShapeDtypeStruct((b, n_e), jnp.float32),
    )(patch2, w_even, w_odd, cb, fcw, fcb)
    return out
